# Initial kernel scaffold; baseline (speedup 1.0000x reference)
import jax
import jax.numpy as jnp
from jax.experimental import pallas as pl


def kernel(x, edge_index, W, a1, b1, a2, b2, out_bias):
    raise NotImplementedError("write your pallas kernel here")


# speedup vs baseline: 1.9189x; 1.9189x over previous
import functools

import jax
import jax.numpy as jnp
from jax.experimental import pallas as pl
from jax.experimental.pallas import tpu as pltpu

N = 10000
D = 128
OUT = 128
BLK = 400


def _proj_body(x_ref, w_ref, a1_ref, a2_ref, b1_ref, b2_ref,
               seq_ref, f1_ref, f2_ref):
    seq = jnp.dot(x_ref[...], w_ref[...], preferred_element_type=jnp.float32)
    seq_ref[...] = seq
    f1_ref[...] = jnp.dot(seq, a1_ref[...]) + b1_ref[0]
    f2_ref[...] = jnp.dot(seq, a2_ref[...]) + b2_ref[0]


def _proj(x2d, W, a1, a2, b1, b2):
    grid = (N // BLK,)
    return pl.pallas_call(
        _proj_body,
        grid=grid,
        in_specs=[
            pl.BlockSpec((BLK, D), lambda i: (i, 0)),
            pl.BlockSpec((D, OUT), lambda i: (0, 0)),
            pl.BlockSpec((OUT, 1), lambda i: (0, 0)),
            pl.BlockSpec((OUT, 1), lambda i: (0, 0)),
            pl.BlockSpec(memory_space=pltpu.SMEM),
            pl.BlockSpec(memory_space=pltpu.SMEM),
        ],
        out_specs=[
            pl.BlockSpec((BLK, OUT), lambda i: (i, 0)),
            pl.BlockSpec((BLK, 1), lambda i: (i, 0)),
            pl.BlockSpec((BLK, 1), lambda i: (i, 0)),
        ],
        out_shape=[
            jax.ShapeDtypeStruct((N, OUT), jnp.float32),
            jax.ShapeDtypeStruct((N, 1), jnp.float32),
            jax.ShapeDtypeStruct((N, 1), jnp.float32),
        ],
    )(x2d, W, a1, a2, b1, b2)


def _combine_body(acc_ref, den_ref, bias_ref, out_ref):
    den = den_ref[...]
    safe = jnp.where(den == 0.0, 1.0, den)
    v = acc_ref[...] / safe + bias_ref[...]
    out_ref[...] = jnp.where(v > 0.0, v, jnp.exp(jnp.minimum(v, 0.0)) - 1.0)


def _combine(acc, den, bias):
    grid = (N // BLK,)
    return pl.pallas_call(
        _combine_body,
        grid=grid,
        in_specs=[
            pl.BlockSpec((BLK, OUT), lambda i: (i, 0)),
            pl.BlockSpec((BLK, 1), lambda i: (i, 0)),
            pl.BlockSpec((1, OUT), lambda i: (0, 0)),
        ],
        out_specs=pl.BlockSpec((BLK, OUT), lambda i: (i, 0)),
        out_shape=jax.ShapeDtypeStruct((N, OUT), jnp.float32),
    )(acc, den, bias)


def kernel(x, edge_index, W, a1, b1, a2, b2, out_bias):
    x2d = jnp.squeeze(x, 0)
    seq, f1, f2 = _proj(x2d, W, a1, a2, b1, b2)
    f1 = f1.reshape(N)
    f2 = f2.reshape(N)
    row = edge_index[0]
    col = edge_index[1]
    logits = f1[row] + f2[col]
    e = jnp.where(logits > 0, logits, 0.2 * logits)
    w = jnp.exp(e)
    den = jax.ops.segment_sum(w, row, num_segments=N)
    acc = jax.ops.segment_sum(w[:, None] * seq[col], row, num_segments=N)
    out = _combine(acc, den.reshape(N, 1), out_bias.reshape(1, OUT))
    return out[None, :, :]


# speedup vs baseline: 16.4119x; 8.5527x over previous
import functools

import jax
import jax.numpy as jnp
from jax import lax
from jax.experimental import pallas as pl
from jax.experimental.pallas import tpu as pltpu
from jax.experimental.pallas import tpu_sc as plsc

N = 10000
D = 128
OUT = 128
BLK = 400

NC = 2
NS = 16
CH = 128
GRP = 8
NCHUNK = 80
EPT = CH * NCHUNK
NPAD = 10112
RPT = NPAD // NS


def _proj_body(x_ref, w_ref, a1_ref, a2_ref, b1_ref, b2_ref,
               seq_ref, f1_ref, f2_ref):
    seq = jnp.dot(x_ref[...], w_ref[...], preferred_element_type=jnp.float32)
    seq_ref[...] = seq
    f1_ref[...] = jnp.dot(seq, a1_ref[...]) + b1_ref[0]
    f2_ref[...] = jnp.dot(seq, a2_ref[...]) + b2_ref[0]


def _proj(x2d, W, a1, a2, b1, b2):
    return pl.pallas_call(
        _proj_body,
        grid=(N // BLK,),
        in_specs=[
            pl.BlockSpec((BLK, D), lambda i: (i, 0)),
            pl.BlockSpec((D, OUT), lambda i: (0, 0)),
            pl.BlockSpec((OUT, 1), lambda i: (0, 0)),
            pl.BlockSpec((OUT, 1), lambda i: (0, 0)),
            pl.BlockSpec(memory_space=pltpu.SMEM),
            pl.BlockSpec(memory_space=pltpu.SMEM),
        ],
        out_specs=[
            pl.BlockSpec((BLK, OUT), lambda i: (i, 0)),
            pl.BlockSpec((BLK, 1), lambda i: (i, 0)),
            pl.BlockSpec((BLK, 1), lambda i: (i, 0)),
        ],
        out_shape=[
            jax.ShapeDtypeStruct((N, OUT), jnp.float32),
            jax.ShapeDtypeStruct((N, 1), jnp.float32),
            jax.ShapeDtypeStruct((N, 1), jnp.float32),
        ],
    )(x2d, W, a1, a2, b1, b2)


def _make_sc():
    mesh = plsc.VectorSubcoreMesh(core_axis_name="c", subcore_axis_name="s",
                                  num_cores=NC, num_subcores=NS)

    drain = [(0, CH), (CH, CH), (2 * CH, CH), (3 * CH, CH), (4 * CH, 120)]

    @functools.partial(
        pl.kernel,
        out_type=(jax.ShapeDtypeStruct((NPAD, OUT), jnp.float32),
                  jax.ShapeDtypeStruct((NPAD, OUT), jnp.float32),
                  jax.ShapeDtypeStruct((NPAD,), jnp.float32),
                  jax.ShapeDtypeStruct((NPAD,), jnp.float32)),
        mesh=mesh,
        scratch_types=[
            pltpu.VMEM((GRP, CH), jnp.int32),
            pltpu.VMEM((GRP, CH), jnp.int32),
            pltpu.VMEM((NPAD,), jnp.float32),
            pltpu.VMEM((NPAD,), jnp.float32),
            pltpu.VMEM((CH, OUT), jnp.float32),
            pltpu.VMEM((CH + 16,), jnp.float32),
            pltpu.VMEM_SHARED((NPAD, OUT), jnp.float32),
            pltpu.VMEM_SHARED((NPAD,), jnp.float32),
            pltpu.SemaphoreType.DMA,
        ],
        compiler_params=pltpu.CompilerParams(needs_layout_passes=False),
    )
    def sc_kernel(seq_hbm, f1_hbm, f2_hbm, rows_hbm, cols_hbm,
                  out0, out1, den0, den1,
                  rowi, coli, f1t, f2t, gbuf, wbuf, acc, dacc, sem):
        c = lax.axis_index("c")
        s = lax.axis_index("s")

        pltpu.sync_copy(f1_hbm, f1t)
        pltpu.sync_copy(f2_hbm, f2t)

        def _zero_row(r, carry):
            for k in range(OUT // 16):
                gbuf[r, pl.ds(k * 16, 16)] = jnp.zeros((16,), jnp.float32)
            return carry
        lax.fori_loop(0, CH, _zero_row, 0)
        for k in range((CH + 16) // 16):
            wbuf[pl.ds(k * 16, 16)] = jnp.zeros((16,), jnp.float32)

        tbase = s * RPT
        for off, sz in drain:
            pltpu.sync_copy(gbuf.at[pl.ds(0, sz)],
                            acc.at[pl.ds(tbase + off, sz)])
            pltpu.sync_copy(wbuf.at[pl.ds(0, sz)],
                            dacc.at[pl.ds(tbase + off, sz)])
        plsc.subcore_barrier()

        def _group(g, carry):
            pltpu.sync_copy(rows_hbm.at[c, s, pl.ds(g * GRP, GRP)], rowi)
            pltpu.sync_copy(cols_hbm.at[c, s, pl.ds(g * GRP, GRP)], coli)
            for j in range(GRP):
                pltpu.async_copy(seq_hbm.at[coli.at[j]], gbuf, sem).wait()
                for k in range(CH // 16):
                    r16 = rowi[j, pl.ds(k * 16, 16)]
                    c16 = coli[j, pl.ds(k * 16, 16)]
                    lg = (plsc.load_gather(f1t, [r16])
                          + plsc.load_gather(f2t, [c16]))
                    w16 = jnp.exp(jnp.where(lg > 0, lg, 0.2 * lg))
                    wbuf[pl.ds(k * 16, 16)] = w16
                def _scale(e, cc):
                    ws = wbuf[pl.ds(e, 16)][0]
                    wv = jnp.full((16,), ws)
                    for kk in range(OUT // 16):
                        gbuf[e, pl.ds(kk * 16, 16)] = (
                            gbuf[e, pl.ds(kk * 16, 16)] * wv)
                    return cc
                lax.fori_loop(0, CH, _scale, 0)
                pltpu.sync_copy(gbuf, acc.at[rowi.at[j]], add=True)
                pltpu.sync_copy(wbuf.at[pl.ds(0, CH)], dacc.at[rowi.at[j]],
                                add=True)
            return carry
        lax.fori_loop(0, NCHUNK // GRP, _group, 0)

        plsc.subcore_barrier()

        @pl.when(c == 0)
        def _():
            for off, sz in drain:
                pltpu.sync_copy(acc.at[pl.ds(tbase + off, sz)],
                                out0.at[pl.ds(tbase + off, sz)])
                pltpu.sync_copy(dacc.at[pl.ds(tbase + off, sz)],
                                wbuf.at[pl.ds(0, sz)])
                pltpu.sync_copy(wbuf.at[pl.ds(0, sz)],
                                den0.at[pl.ds(tbase + off, sz)])

        @pl.when(c == 1)
        def _():
            for off, sz in drain:
                pltpu.sync_copy(acc.at[pl.ds(tbase + off, sz)],
                                out1.at[pl.ds(tbase + off, sz)])
                pltpu.sync_copy(dacc.at[pl.ds(tbase + off, sz)],
                                wbuf.at[pl.ds(0, sz)])
                pltpu.sync_copy(wbuf.at[pl.ds(0, sz)],
                                den1.at[pl.ds(tbase + off, sz)])

    return sc_kernel


_sc_kernel = _make_sc()


def _combine_body(a0_ref, a1_ref, d0_ref, d1_ref, bias_ref, out_ref):
    den = d0_ref[...] + d1_ref[...]
    safe = jnp.where(den == 0.0, 1.0, den)
    v = (a0_ref[...] + a1_ref[...]) / safe + bias_ref[...]
    out_ref[...] = jnp.where(v > 0.0, v, jnp.exp(jnp.minimum(v, 0.0)) - 1.0)


def _combine(p0, p1, d0, d1, bias):
    return pl.pallas_call(
        _combine_body,
        grid=(N // BLK,),
        in_specs=[
            pl.BlockSpec((BLK, OUT), lambda i: (i, 0)),
            pl.BlockSpec((BLK, OUT), lambda i: (i, 0)),
            pl.BlockSpec((BLK, 1), lambda i: (i, 0)),
            pl.BlockSpec((BLK, 1), lambda i: (i, 0)),
            pl.BlockSpec((1, OUT), lambda i: (0, 0)),
        ],
        out_specs=pl.BlockSpec((BLK, OUT), lambda i: (i, 0)),
        out_shape=jax.ShapeDtypeStruct((N, OUT), jnp.float32),
    )(p0, p1, d0, d1, bias)


def kernel(x, edge_index, W, a1, b1, a2, b2, out_bias):
    x2d = jnp.squeeze(x, 0)
    seq, f1, f2 = _proj(x2d, W, a1, a2, b1, b2)
    zpad = jnp.zeros((NPAD - N,), jnp.float32)
    f1p = jnp.concatenate([f1.reshape(N), zpad])
    f2p = jnp.concatenate([f2.reshape(N), zpad])

    rows = edge_index[0].reshape(32, -1)
    cols = edge_index[1].reshape(32, -1)
    dpad = EPT - rows.shape[1]
    drow = jnp.broadcast_to(
        N + jnp.arange(dpad, dtype=jnp.int32) % (NPAD - N), (32, dpad))
    dcol = jnp.zeros((32, dpad), jnp.int32)
    rows3d = jnp.concatenate([rows, drow], axis=1).reshape(NC, NS, NCHUNK, CH)
    cols3d = jnp.concatenate([cols, dcol], axis=1).reshape(NC, NS, NCHUNK, CH)

    p0, p1, d0, d1 = _sc_kernel(seq, f1p, f2p, rows3d, cols3d)
    out = _combine(p0, p1, d0.reshape(NPAD, 1), d1.reshape(NPAD, 1),
                   out_bias.reshape(1, OUT))
    return out[None, :, :]
